# baseline (speedup 1.0000x reference)
import functools
import math

import jax
import jax.numpy as jnp
from jax import lax
from jax.experimental import pallas as pl
from jax.experimental.pallas import tpu as pltpu
from jax.experimental.pallas import tpu_sc as plsc

N = 10000
E = 320000
F = 128
NC = 2
NS = 16
NW = NC * NS
C = 125
NCH = (E // NW) // C
CL = 125
NCL = (E // NW) // CL
D = 2
GR = 80
NGRP = N // GR
NGT = -(-NGRP // NS)
DEGW = 128
NBUF = 2
RB = 400
BNS = 1.0 / math.sqrt(1.0 + 1e-5)


def _fill2d(ref, nrows, ncols, value):
    v = jnp.full((16,), value, jnp.float32)

    def body(j, carry):
        for k in range(ncols // 16):
            ref[j, pl.ds(k * 16, 16)] = v
        return carry

    lax.fori_loop(0, nrows, body, 0)


def _deg_body(rows_hbm, out_hbm, rows_v, obuf, acc_sh, *ssems):
    cid = lax.axis_index("c")
    sid = lax.axis_index("s")
    wid = sid * NC + cid
    pltpu.sync_copy(rows_hbm.at[wid], rows_v)
    _fill2d(obuf, C, DEGW, 0.0)

    def zbody(t, carry):
        g = sid + NS * t

        @pl.when(g < NGRP)
        def _():
            pltpu.sync_copy(obuf.at[pl.ds(0, GR)], acc_sh.at[pl.ds(g * GR, GR)])

        return carry

    lax.fori_loop(0, NGT, zbody, 0)
    _fill2d(obuf, C, DEGW, 1.0)
    plsc.subcore_barrier()

    def body(t, carry):
        j0 = NBUF * t
        for k in range(NBUF):
            j = j0 + k

            @pl.when(t > 0)
            def _(k=k):
                pltpu.make_async_copy(obuf, acc_sh.at[rows_v.at[j - NBUF]],
                                      ssems[k]).wait()

            pltpu.async_copy(obuf, acc_sh.at[rows_v.at[j]], ssems[k],
                             add=True)
        return carry

    lax.fori_loop(0, NCH // NBUF, body, 0)
    for k in range(NBUF):
        pltpu.make_async_copy(obuf, acc_sh.at[rows_v.at[NCH - NBUF + k]],
                              ssems[k]).wait()
    plsc.subcore_barrier()

    def obody(t, carry):
        g = sid + NS * t

        @pl.when(g < NGRP)
        def _():
            pltpu.sync_copy(acc_sh.at[pl.ds(g * GR, GR)],
                            out_hbm.at[cid, pl.ds(g * GR, GR)])

        return carry

    lax.fori_loop(0, NGT, obody, 0)


_deg = pl.kernel(
    _deg_body,
    out_type=jax.ShapeDtypeStruct((NC, N, DEGW), jnp.float32),
    mesh=plsc.VectorSubcoreMesh(core_axis_name="c", subcore_axis_name="s"),
    scratch_types=[
        pltpu.VMEM((NCH, C), jnp.int32),
        pltpu.VMEM((C, DEGW), jnp.float32),
        pltpu.VMEM_SHARED((N, DEGW), jnp.float32),
    ] + [pltpu.SemaphoreType.DMA] * NBUF,
)


def _lap_body(rows_hbm, cols_hbm, u_hbm, out_hbm, rows_v, cibuf, gb, acc_sh,
              *sems):
    isems = sems[:D]
    gsems = sems[D:2 * D]
    ssems = sems[2 * D:]
    cid = lax.axis_index("c")
    sid = lax.axis_index("s")
    wid = sid * NC + cid
    ch = cols_hbm.at[wid]
    pltpu.sync_copy(rows_hbm.at[wid], rows_v)
    _fill2d(gb.at[0], CL, F, 0.0)

    def zbody(t, carry):
        g = sid + NS * t

        @pl.when(g < NGRP)
        def _():
            pltpu.sync_copy(gb.at[0, pl.ds(0, GR)], acc_sh.at[pl.ds(g * GR, GR)])

        return carry

    lax.fori_loop(0, NGT, zbody, 0)
    for k in range(D):
        pltpu.async_copy(ch.at[k], cibuf.at[k], isems[k])
    plsc.subcore_barrier()

    def body(t, carry):
        j0 = D * t
        for k in range(D):
            j = j0 + k
            pltpu.make_async_copy(ch.at[j], cibuf.at[k], isems[k]).wait()

            @pl.when(t > 0)
            def _(k=k, j=j):
                pltpu.make_async_copy(gb.at[k], acc_sh.at[rows_v.at[j - D]],
                                      ssems[k]).wait()

            pltpu.async_copy(u_hbm.at[cibuf.at[k, 0]], gb.at[k], gsems[k])
        for k in range(D):
            j = j0 + k
            pltpu.make_async_copy(u_hbm.at[cibuf.at[k, 0]], gb.at[k],
                                  gsems[k]).wait()

            @pl.when(j + D < NCL)
            def _(k=k, j=j):
                pltpu.async_copy(ch.at[j + D], cibuf.at[k], isems[k])

            pltpu.async_copy(gb.at[k], acc_sh.at[rows_v.at[j]], ssems[k],
                             add=True)
        return carry

    lax.fori_loop(0, NCL // D, body, 0)
    for k in range(D):
        pltpu.make_async_copy(gb.at[k], acc_sh.at[rows_v.at[NCL - D + k]],
                              ssems[k]).wait()
    plsc.subcore_barrier()

    def obody(t, carry):
        g = sid + NS * t

        @pl.when(g < NGRP)
        def _():
            pltpu.sync_copy(acc_sh.at[pl.ds(g * GR, GR)],
                            out_hbm.at[cid, pl.ds(g * GR, GR)])

        return carry

    lax.fori_loop(0, NGT, obody, 0)


_lap = pl.kernel(
    _lap_body,
    out_type=jax.ShapeDtypeStruct((NC, N, F), jnp.float32),
    mesh=plsc.VectorSubcoreMesh(core_axis_name="c", subcore_axis_name="s"),
    scratch_types=[
        pltpu.VMEM((NCL, CL), jnp.int32),
        pltpu.VMEM((D, 1, CL), jnp.int32),
        pltpu.VMEM((D, CL, F), jnp.float32),
        pltpu.VMEM_SHARED((N, F), jnp.float32),
    ] + [pltpu.SemaphoreType.DMA] * (3 * D),
)


_row_spec = pl.BlockSpec((RB, F), lambda i: (i, 0))
_s_spec = pl.BlockSpec((NC, RB, F), lambda i: (0, i, 0))
_w_spec = pl.BlockSpec((F, F), lambda i: (0, 0))
_b_spec = pl.BlockSpec((1, F), lambda i: (0, 0))
_GRID = (N // RB,)
_f32 = jnp.float32


def _cat(s_ref):
    return s_ref[0] + s_ref[1]


def _degfin_body(s_ref, dis_ref):
    d = s_ref[0, :, 0:1] + s_ref[1, :, 0:1]
    dis = jnp.where(d > 0, lax.rsqrt(jnp.maximum(d, 1.0)), 0.0)
    dis_ref[...] = jnp.broadcast_to(dis, dis_ref.shape)


def _degfin(deg_s):
    return pl.pallas_call(
        _degfin_body,
        out_shape=jax.ShapeDtypeStruct((N, F), _f32),
    )(deg_s)


def _pre_body(h_ref, dis_ref, w_ref, u_ref, acc_ref):
    h = h_ref[...]
    u_ref[...] = dis_ref[...] * h
    acc_ref[...] = jnp.dot(h, w_ref[...], preferred_element_type=_f32)


def _pre(h, dis, w):
    return pl.pallas_call(
        _pre_body, grid=_GRID,
        in_specs=[_row_spec, _row_spec, _w_spec],
        out_specs=[_row_spec, _row_spec],
        out_shape=[jax.ShapeDtypeStruct((N, F), _f32)] * 2,
    )(h, dis, w)


def _mid1_body(s_ref, dis_ref, w_ref, acc_ref, tx_ref, u_ref, out_ref):
    dis = dis_ref[...]
    tx = -dis * _cat(s_ref)
    tx_ref[...] = tx
    u_ref[...] = dis * tx
    out_ref[...] = acc_ref[...] + jnp.dot(tx, w_ref[...],
                                          preferred_element_type=_f32)


def _mid1(s, dis, w, acc):
    return pl.pallas_call(
        _mid1_body, grid=_GRID,
        in_specs=[_s_spec, _row_spec, _w_spec, _row_spec],
        out_specs=[_row_spec] * 3,
        out_shape=[jax.ShapeDtypeStruct((N, F), _f32)] * 3,
    )(s, dis, w, acc)


def _mid2_body(s_ref, dis_ref, txm2_ref, w_ref, acc_ref, tx_ref, u_ref,
               out_ref):
    dis = dis_ref[...]
    tx = -2.0 * dis * _cat(s_ref) - txm2_ref[...]
    tx_ref[...] = tx
    u_ref[...] = dis * tx
    out_ref[...] = acc_ref[...] + jnp.dot(tx, w_ref[...],
                                          preferred_element_type=_f32)


def _mid2(s, dis, txm2, w, acc):
    return pl.pallas_call(
        _mid2_body, grid=_GRID,
        in_specs=[_s_spec, _row_spec, _row_spec, _w_spec, _row_spec],
        out_specs=[_row_spec] * 3,
        out_shape=[jax.ShapeDtypeStruct((N, F), _f32)] * 3,
    )(s, dis, txm2, w, acc)


def _fin_body(s_ref, dis_ref, txm2_ref, w_ref, acc_ref, cb_ref, g_ref,
              be_ref, h_ref):
    tx = -2.0 * dis_ref[...] * _cat(s_ref) - txm2_ref[...]
    acc = acc_ref[...] + jnp.dot(tx, w_ref[...], preferred_element_type=_f32)
    h_ref[...] = jnp.maximum((acc + cb_ref[...]) * BNS * g_ref[...]
                             + be_ref[...], 0.0)


def _fin(s, dis, txm2, w, acc, cb, g, be):
    return pl.pallas_call(
        _fin_body, grid=_GRID,
        in_specs=[_s_spec, _row_spec, _row_spec, _w_spec, _row_spec,
                  _b_spec, _b_spec, _b_spec],
        out_specs=_row_spec,
        out_shape=jax.ShapeDtypeStruct((N, F), _f32),
    )(s, dis, txm2, w, acc, cb, g, be)


def _fin3_body(s_ref, dis_ref, txm2_ref, w_ref, acc_ref, cb_ref, g_ref,
               be_ref, hw_ref, hb_ref, o_ref):
    tx = -2.0 * dis_ref[...] * _cat(s_ref) - txm2_ref[...]
    acc = acc_ref[...] + jnp.dot(tx, w_ref[...], preferred_element_type=_f32)
    h = jnp.maximum((acc + cb_ref[...]) * BNS * g_ref[...] + be_ref[...], 0.0)
    o_ref[...] = jnp.dot(h, hw_ref[...], preferred_element_type=_f32) \
        + hb_ref[...]


def _fin3(s, dis, txm2, w, acc, cb, g, be, hw, hb):
    return pl.pallas_call(
        _fin3_body, grid=_GRID,
        in_specs=[_s_spec, _row_spec, _row_spec, _w_spec, _row_spec,
                  _b_spec, _b_spec, _b_spec, _w_spec, _b_spec],
        out_specs=_row_spec,
        out_shape=jax.ShapeDtypeStruct((N, F), _f32),
    )(s, dis, txm2, w, acc, cb, g, be, hw, hb)


def _layer(h, rc, dis, w, cb, g, be, head=None):
    u, acc = _pre(h, dis, w[0])
    s = _lap(rc[0], rc[1], u)
    tx1, u, acc = _mid1(s, dis, w[1], acc)
    s = _lap(rc[0], rc[1], u)
    tx2, u, acc = _mid2(s, dis, h, w[2], acc)
    s = _lap(rc[0], rc[1], u)
    if head is None:
        return _fin(s, dis, tx1, w[3], acc, cb, g, be)
    return _fin3(s, dis, tx1, w[3], acc, cb, g, be, head[0], head[1])


def kernel(x, ei, W1, cb1, W2, cb2, W3, cb3, g1, be1, g2, be2, g3, be3,
           headW, headb):
    rows_d = ei[0].reshape(NW, NCH, C)
    rows = ei[0].reshape(NW, NCL, CL)
    cols = ei[1].reshape(NW, NCL, 1, CL)
    deg_s = _deg(rows_d)
    dis = _degfin(deg_s)
    r2 = lambda v: v.reshape(1, F)
    h = _layer(x, (rows, cols), dis, W1, r2(cb1), r2(g1), r2(be1))
    h = _layer(h, (rows, cols), dis, W2, r2(cb2), r2(g2), r2(be2))
    return _layer(h, (rows, cols), dis, W3, r2(cb3), r2(g3), r2(be3),
                  head=(headW, r2(headb)))


# speedup vs baseline: 1.0420x; 1.0420x over previous
import functools
import math

import jax
import jax.numpy as jnp
from jax import lax
from jax.experimental import pallas as pl
from jax.experimental.pallas import tpu as pltpu
from jax.experimental.pallas import tpu_sc as plsc

N = 10000
E = 320000
F = 128
NC = 2
NS = 16
NW = NC * NS
C = 125
NCH = (E // NW) // C
GR = 80
NGRP = N // GR
NGT = -(-NGRP // NS)
DEGW = 128
NBUF = 2
RB = 400
BNS = 1.0 / math.sqrt(1.0 + 1e-5)


def _fill2d(ref, nrows, ncols, value):
    v = jnp.full((16,), value, jnp.float32)

    def body(j, carry):
        for k in range(ncols // 16):
            ref[j, pl.ds(k * 16, 16)] = v
        return carry

    lax.fori_loop(0, nrows, body, 0)


def _deg_body(rows_hbm, out_hbm, rows_v, obuf, acc_sh, *ssems):
    cid = lax.axis_index("c")
    sid = lax.axis_index("s")
    wid = sid * NC + cid
    pltpu.sync_copy(rows_hbm.at[wid], rows_v)
    _fill2d(obuf, C, DEGW, 0.0)

    def zbody(t, carry):
        g = sid + NS * t

        @pl.when(g < NGRP)
        def _():
            pltpu.sync_copy(obuf.at[pl.ds(0, GR)], acc_sh.at[pl.ds(g * GR, GR)])

        return carry

    lax.fori_loop(0, NGT, zbody, 0)
    _fill2d(obuf, C, DEGW, 1.0)
    plsc.subcore_barrier()

    def body(t, carry):
        j0 = NBUF * t
        for k in range(NBUF):
            j = j0 + k

            @pl.when(t > 0)
            def _(k=k):
                pltpu.make_async_copy(obuf, acc_sh.at[rows_v.at[j - NBUF]],
                                      ssems[k]).wait()

            pltpu.async_copy(obuf, acc_sh.at[rows_v.at[j]], ssems[k],
                             add=True)
        return carry

    lax.fori_loop(0, NCH // NBUF, body, 0)
    for k in range(NBUF):
        pltpu.make_async_copy(obuf, acc_sh.at[rows_v.at[NCH - NBUF + k]],
                              ssems[k]).wait()
    plsc.subcore_barrier()

    def obody(t, carry):
        g = sid + NS * t

        @pl.when(g < NGRP)
        def _():
            pltpu.sync_copy(acc_sh.at[pl.ds(g * GR, GR)],
                            out_hbm.at[cid, pl.ds(g * GR, GR)])

        return carry

    lax.fori_loop(0, NGT, obody, 0)


_deg = pl.kernel(
    _deg_body,
    out_type=jax.ShapeDtypeStruct((NC, N, DEGW), jnp.float32),
    mesh=plsc.VectorSubcoreMesh(core_axis_name="c", subcore_axis_name="s"),
    scratch_types=[
        pltpu.VMEM((NCH, C), jnp.int32),
        pltpu.VMEM((C, DEGW), jnp.float32),
        pltpu.VMEM_SHARED((N, DEGW), jnp.float32),
    ] + [pltpu.SemaphoreType.DMA] * NBUF,
)


def _lap_body(ei2_hbm, u_hbm, out_hbm, ibuf, gb, acc_sh, *sems):
    isems = sems[:NBUF]
    gsems = sems[NBUF:]
    cid = lax.axis_index("c")
    sid = lax.axis_index("s")
    wid = sid * NC + cid
    eh = ei2_hbm.at[wid]
    _fill2d(gb.at[0], C, F, 0.0)

    def zbody(t, carry):
        g = sid + NS * t

        @pl.when(g < NGRP)
        def _():
            pltpu.sync_copy(gb.at[0, pl.ds(0, GR)], acc_sh.at[pl.ds(g * GR, GR)])

        return carry

    lax.fori_loop(0, NGT, zbody, 0)
    for b in range(NBUF):
        pltpu.async_copy(eh.at[b], ibuf.at[b], isems[b])
    plsc.subcore_barrier()

    def body(t, carry):
        j0 = NBUF * t
        for b in range(NBUF):
            j = j0 + b
            pltpu.make_async_copy(eh.at[j], ibuf.at[b], isems[b]).wait()
            pltpu.async_copy(u_hbm.at[ibuf.at[b, 1]], gb.at[b], gsems[b])
        for b in range(NBUF):
            j = j0 + b
            pltpu.make_async_copy(u_hbm.at[ibuf.at[b, 1]], gb.at[b],
                                  gsems[b]).wait()
            pltpu.sync_copy(gb.at[b], acc_sh.at[ibuf.at[b, 0]], add=True)

            @pl.when(j + NBUF < NCH)
            def _(j=j, b=b):
                pltpu.async_copy(eh.at[j + NBUF], ibuf.at[b], isems[b])

        return carry

    lax.fori_loop(0, NCH // NBUF, body, 0)
    plsc.subcore_barrier()

    def obody(t, carry):
        g = sid + NS * t

        @pl.when(g < NGRP)
        def _():
            pltpu.sync_copy(acc_sh.at[pl.ds(g * GR, GR)],
                            out_hbm.at[cid, pl.ds(g * GR, GR)])

        return carry

    lax.fori_loop(0, NGT, obody, 0)


_lap = pl.kernel(
    _lap_body,
    out_type=jax.ShapeDtypeStruct((NC, N, F), jnp.float32),
    mesh=plsc.VectorSubcoreMesh(core_axis_name="c", subcore_axis_name="s"),
    scratch_types=[
        pltpu.VMEM((NBUF, 2, C), jnp.int32),
        pltpu.VMEM((NBUF, C, F), jnp.float32),
        pltpu.VMEM_SHARED((N, F), jnp.float32),
    ] + [pltpu.SemaphoreType.DMA] * (2 * NBUF),
)


_row_spec = pl.BlockSpec((RB, F), lambda i: (i, 0))
_s_spec = pl.BlockSpec((NC, RB, F), lambda i: (0, i, 0))
_w_spec = pl.BlockSpec((F, F), lambda i: (0, 0))
_b_spec = pl.BlockSpec((1, F), lambda i: (0, 0))
_GRID = (N // RB,)
_f32 = jnp.float32


def _cat(s_ref):
    return s_ref[0] + s_ref[1]


def _degfin_body(s_ref, dis_ref):
    d = s_ref[0, :, 0:1] + s_ref[1, :, 0:1]
    dis = jnp.where(d > 0, lax.rsqrt(jnp.maximum(d, 1.0)), 0.0)
    dis_ref[...] = jnp.broadcast_to(dis, dis_ref.shape)


def _degfin(deg_s):
    return pl.pallas_call(
        _degfin_body,
        out_shape=jax.ShapeDtypeStruct((N, F), _f32),
    )(deg_s)


def _pre_body(h_ref, dis_ref, w_ref, u_ref, acc_ref):
    h = h_ref[...]
    u_ref[...] = dis_ref[...] * h
    acc_ref[...] = jnp.dot(h, w_ref[...], preferred_element_type=_f32)


def _pre(h, dis, w):
    return pl.pallas_call(
        _pre_body, grid=_GRID,
        in_specs=[_row_spec, _row_spec, _w_spec],
        out_specs=[_row_spec, _row_spec],
        out_shape=[jax.ShapeDtypeStruct((N, F), _f32)] * 2,
    )(h, dis, w)


def _mid1_body(s_ref, dis_ref, w_ref, acc_ref, tx_ref, u_ref, out_ref):
    dis = dis_ref[...]
    tx = -dis * _cat(s_ref)
    tx_ref[...] = tx
    u_ref[...] = dis * tx
    out_ref[...] = acc_ref[...] + jnp.dot(tx, w_ref[...],
                                          preferred_element_type=_f32)


def _mid1(s, dis, w, acc):
    return pl.pallas_call(
        _mid1_body, grid=_GRID,
        in_specs=[_s_spec, _row_spec, _w_spec, _row_spec],
        out_specs=[_row_spec] * 3,
        out_shape=[jax.ShapeDtypeStruct((N, F), _f32)] * 3,
    )(s, dis, w, acc)


def _mid2_body(s_ref, dis_ref, txm2_ref, w_ref, acc_ref, tx_ref, u_ref,
               out_ref):
    dis = dis_ref[...]
    tx = -2.0 * dis * _cat(s_ref) - txm2_ref[...]
    tx_ref[...] = tx
    u_ref[...] = dis * tx
    out_ref[...] = acc_ref[...] + jnp.dot(tx, w_ref[...],
                                          preferred_element_type=_f32)


def _mid2(s, dis, txm2, w, acc):
    return pl.pallas_call(
        _mid2_body, grid=_GRID,
        in_specs=[_s_spec, _row_spec, _row_spec, _w_spec, _row_spec],
        out_specs=[_row_spec] * 3,
        out_shape=[jax.ShapeDtypeStruct((N, F), _f32)] * 3,
    )(s, dis, txm2, w, acc)


def _fin_body(s_ref, dis_ref, txm2_ref, w_ref, acc_ref, cb_ref, g_ref,
              be_ref, h_ref):
    tx = -2.0 * dis_ref[...] * _cat(s_ref) - txm2_ref[...]
    acc = acc_ref[...] + jnp.dot(tx, w_ref[...], preferred_element_type=_f32)
    h_ref[...] = jnp.maximum((acc + cb_ref[...]) * BNS * g_ref[...]
                             + be_ref[...], 0.0)


def _fin(s, dis, txm2, w, acc, cb, g, be):
    return pl.pallas_call(
        _fin_body, grid=_GRID,
        in_specs=[_s_spec, _row_spec, _row_spec, _w_spec, _row_spec,
                  _b_spec, _b_spec, _b_spec],
        out_specs=_row_spec,
        out_shape=jax.ShapeDtypeStruct((N, F), _f32),
    )(s, dis, txm2, w, acc, cb, g, be)


def _fin3_body(s_ref, dis_ref, txm2_ref, w_ref, acc_ref, cb_ref, g_ref,
               be_ref, hw_ref, hb_ref, o_ref):
    tx = -2.0 * dis_ref[...] * _cat(s_ref) - txm2_ref[...]
    acc = acc_ref[...] + jnp.dot(tx, w_ref[...], preferred_element_type=_f32)
    h = jnp.maximum((acc + cb_ref[...]) * BNS * g_ref[...] + be_ref[...], 0.0)
    o_ref[...] = jnp.dot(h, hw_ref[...], preferred_element_type=_f32) \
        + hb_ref[...]


def _fin3(s, dis, txm2, w, acc, cb, g, be, hw, hb):
    return pl.pallas_call(
        _fin3_body, grid=_GRID,
        in_specs=[_s_spec, _row_spec, _row_spec, _w_spec, _row_spec,
                  _b_spec, _b_spec, _b_spec, _w_spec, _b_spec],
        out_specs=_row_spec,
        out_shape=jax.ShapeDtypeStruct((N, F), _f32),
    )(s, dis, txm2, w, acc, cb, g, be, hw, hb)


def _layer(h, rc, dis, w, cb, g, be, head=None):
    u, acc = _pre(h, dis, w[0])
    s = _lap(rc, u)
    tx1, u, acc = _mid1(s, dis, w[1], acc)
    s = _lap(rc, u)
    tx2, u, acc = _mid2(s, dis, h, w[2], acc)
    s = _lap(rc, u)
    if head is None:
        return _fin(s, dis, tx1, w[3], acc, cb, g, be)
    return _fin3(s, dis, tx1, w[3], acc, cb, g, be, head[0], head[1])


def kernel(x, ei, W1, cb1, W2, cb2, W3, cb3, g1, be1, g2, be2, g3, be3,
           headW, headb):
    rows = ei[0].reshape(NW, NCH, C)
    cols = ei[1].reshape(NW, NCH, C)
    ei2 = jnp.stack([rows, cols], axis=2)
    deg_s = _deg(rows)
    dis = _degfin(deg_s)
    r2 = lambda v: v.reshape(1, F)
    h = _layer(x, ei2, dis, W1, r2(cb1), r2(g1), r2(be1))
    h = _layer(h, ei2, dis, W2, r2(cb2), r2(g2), r2(be2))
    return _layer(h, ei2, dis, W3, r2(cb3), r2(g3), r2(be3),
                  head=(headW, r2(headb)))


# speedup vs baseline: 1.0647x; 1.0218x over previous
import functools
import math

import jax
import jax.numpy as jnp
from jax import lax
from jax.experimental import pallas as pl
from jax.experimental.pallas import tpu as pltpu
from jax.experimental.pallas import tpu_sc as plsc

N = 10000
E = 320000
F = 128
NC = 2
NS = 16
NW = NC * NS
C = 125
NCH = (E // NW) // C
GR = 80
NGRP = N // GR
NGT = -(-NGRP // NS)
DEGW = 128
NBUF = 2
RB = 400
BNS = 1.0 / math.sqrt(1.0 + 1e-5)


def _fill2d(ref, nrows, ncols, value):
    v = jnp.full((16,), value, jnp.float32)

    def body(j, carry):
        for k in range(ncols // 16):
            ref[j, pl.ds(k * 16, 16)] = v
        return carry

    lax.fori_loop(0, nrows, body, 0)


def _deg_body(rows_hbm, out_hbm, rows_v, obuf, acc_sh, *ssems):
    cid = lax.axis_index("c")
    sid = lax.axis_index("s")
    wid = sid * NC + cid
    pltpu.sync_copy(rows_hbm.at[wid], rows_v)
    _fill2d(obuf, C, DEGW, 0.0)

    def zbody(t, carry):
        g = sid + NS * t

        @pl.when(g < NGRP)
        def _():
            pltpu.sync_copy(obuf.at[pl.ds(0, GR)], acc_sh.at[pl.ds(g * GR, GR)])

        return carry

    lax.fori_loop(0, NGT, zbody, 0)
    _fill2d(obuf, C, DEGW, 1.0)
    plsc.subcore_barrier()

    def body(t, carry):
        j0 = NBUF * t
        for k in range(NBUF):
            j = j0 + k

            @pl.when(t > 0)
            def _(k=k):
                pltpu.make_async_copy(obuf, acc_sh.at[rows_v.at[j - NBUF]],
                                      ssems[k]).wait()

            pltpu.async_copy(obuf, acc_sh.at[rows_v.at[j]], ssems[k],
                             add=True)
        return carry

    lax.fori_loop(0, NCH // NBUF, body, 0)
    for k in range(NBUF):
        pltpu.make_async_copy(obuf, acc_sh.at[rows_v.at[NCH - NBUF + k]],
                              ssems[k]).wait()
    plsc.subcore_barrier()

    def obody(t, carry):
        g = sid + NS * t

        @pl.when(g < NGRP)
        def _():
            pltpu.sync_copy(acc_sh.at[pl.ds(g * GR, GR)],
                            out_hbm.at[cid, pl.ds(g * GR, GR)])

        return carry

    lax.fori_loop(0, NGT, obody, 0)


_deg = pl.kernel(
    _deg_body,
    out_type=jax.ShapeDtypeStruct((NC, N, DEGW), jnp.float32),
    mesh=plsc.VectorSubcoreMesh(core_axis_name="c", subcore_axis_name="s"),
    scratch_types=[
        pltpu.VMEM((NCH, C), jnp.int32),
        pltpu.VMEM((C, DEGW), jnp.float32),
        pltpu.VMEM_SHARED((N, DEGW), jnp.float32),
    ] + [pltpu.SemaphoreType.DMA] * NBUF,
)


def _lap_body(ei2_hbm, u_hbm, out_hbm, ibuf, gb, acc_sh, *sems):
    isems = sems[:NBUF]
    gsems = sems[NBUF:]
    cid = lax.axis_index("c")
    sid = lax.axis_index("s")
    wid = sid * NC + cid
    eh = ei2_hbm.at[wid]
    _fill2d(gb.at[0], C, F, 0.0)

    def zbody(t, carry):
        g = sid + NS * t

        @pl.when(g < NGRP)
        def _():
            pltpu.sync_copy(gb.at[0, pl.ds(0, GR)], acc_sh.at[pl.ds(g * GR, GR)])

        return carry

    lax.fori_loop(0, NGT, zbody, 0)
    for b in range(NBUF):
        pltpu.async_copy(eh.at[b], ibuf.at[b], isems[b])
    plsc.subcore_barrier()

    def body(t, carry):
        j0 = NBUF * t
        for b in range(NBUF):
            j = j0 + b
            pltpu.make_async_copy(eh.at[j], ibuf.at[b], isems[b]).wait()
            pltpu.async_copy(u_hbm.at[ibuf.at[b, 1]], gb.at[b], gsems[b])
        for b in range(NBUF):
            j = j0 + b
            pltpu.make_async_copy(u_hbm.at[ibuf.at[b, 1]], gb.at[b],
                                  gsems[b]).wait()
            pltpu.sync_copy(gb.at[b], acc_sh.at[ibuf.at[b, 0]], add=True)

            @pl.when(j + NBUF < NCH)
            def _(j=j, b=b):
                pltpu.async_copy(eh.at[j + NBUF], ibuf.at[b], isems[b])

        return carry

    lax.fori_loop(0, NCH // NBUF, body, 0)
    plsc.subcore_barrier()

    def obody(t, carry):
        g = sid + NS * t

        @pl.when(g < NGRP)
        def _():
            pltpu.sync_copy(acc_sh.at[pl.ds(g * GR, GR)],
                            out_hbm.at[cid, pl.ds(g * GR, GR)])

        return carry

    lax.fori_loop(0, NGT, obody, 0)


_lap = pl.kernel(
    _lap_body,
    out_type=jax.ShapeDtypeStruct((NC, N, F), jnp.float32),
    mesh=plsc.VectorSubcoreMesh(core_axis_name="c", subcore_axis_name="s"),
    scratch_types=[
        pltpu.VMEM((NBUF, 2, C), jnp.int32),
        pltpu.VMEM((NBUF, C, F), jnp.float32),
        pltpu.VMEM_SHARED((N, F), jnp.float32),
    ] + [pltpu.SemaphoreType.DMA] * (2 * NBUF),
)


_row_spec = pl.BlockSpec((RB, F), lambda i: (i, 0))
_s_spec = pl.BlockSpec((NC, RB, F), lambda i: (0, i, 0))
_w_spec = pl.BlockSpec((F, F), lambda i: (0, 0))
_b_spec = pl.BlockSpec((1, F), lambda i: (0, 0))
_GRID = (N // RB,)
_f32 = jnp.float32


def _cat(s_ref):
    return s_ref[0] + s_ref[1]


def _degfin_body(s_ref, dis_ref):
    d = s_ref[0, :, 0:1] + s_ref[1, :, 0:1]
    dis = jnp.where(d > 0, lax.rsqrt(jnp.maximum(d, 1.0)), 0.0)
    dis_ref[...] = jnp.broadcast_to(dis, dis_ref.shape)


def _degfin(deg_s):
    return pl.pallas_call(
        _degfin_body,
        out_shape=jax.ShapeDtypeStruct((N, F), _f32),
    )(deg_s)


def _preu_body(h_ref, dis_ref, u_ref):
    u_ref[...] = dis_ref[...] * h_ref[...]


def _preu(h, dis):
    return pl.pallas_call(
        _preu_body, grid=_GRID,
        in_specs=[_row_spec, _row_spec],
        out_specs=_row_spec,
        out_shape=jax.ShapeDtypeStruct((N, F), _f32),
    )(h, dis)


def _mm_body(h_ref, w_ref, acc_ref):
    acc_ref[...] = jnp.dot(h_ref[...], w_ref[...], preferred_element_type=_f32)


def _mm(h, w):
    return pl.pallas_call(
        _mm_body, grid=_GRID,
        in_specs=[_row_spec, _w_spec],
        out_specs=_row_spec,
        out_shape=jax.ShapeDtypeStruct((N, F), _f32),
    )(h, w)


def _mma_body(acc_ref, tx_ref, w_ref, out_ref):
    out_ref[...] = acc_ref[...] + jnp.dot(tx_ref[...], w_ref[...],
                                          preferred_element_type=_f32)


def _mma(acc, tx, w):
    return pl.pallas_call(
        _mma_body, grid=_GRID,
        in_specs=[_row_spec, _row_spec, _w_spec],
        out_specs=_row_spec,
        out_shape=jax.ShapeDtypeStruct((N, F), _f32),
    )(acc, tx, w)


def _midu1_body(s_ref, dis_ref, tx_ref, u_ref):
    dis = dis_ref[...]
    tx = -dis * _cat(s_ref)
    tx_ref[...] = tx
    u_ref[...] = dis * tx


def _midu1(s, dis):
    return pl.pallas_call(
        _midu1_body, grid=_GRID,
        in_specs=[_s_spec, _row_spec],
        out_specs=[_row_spec] * 2,
        out_shape=[jax.ShapeDtypeStruct((N, F), _f32)] * 2,
    )(s, dis)


def _midu2_body(s_ref, dis_ref, txm2_ref, tx_ref, u_ref):
    dis = dis_ref[...]
    tx = -2.0 * dis * _cat(s_ref) - txm2_ref[...]
    tx_ref[...] = tx
    u_ref[...] = dis * tx


def _midu2(s, dis, txm2):
    return pl.pallas_call(
        _midu2_body, grid=_GRID,
        in_specs=[_s_spec, _row_spec, _row_spec],
        out_specs=[_row_spec] * 2,
        out_shape=[jax.ShapeDtypeStruct((N, F), _f32)] * 2,
    )(s, dis, txm2)


def _mid1_body(s_ref, dis_ref, w_ref, acc_ref, tx_ref, u_ref, out_ref):
    dis = dis_ref[...]
    tx = -dis * _cat(s_ref)
    tx_ref[...] = tx
    u_ref[...] = dis * tx
    out_ref[...] = acc_ref[...] + jnp.dot(tx, w_ref[...],
                                          preferred_element_type=_f32)


def _mid1(s, dis, w, acc):
    return pl.pallas_call(
        _mid1_body, grid=_GRID,
        in_specs=[_s_spec, _row_spec, _w_spec, _row_spec],
        out_specs=[_row_spec] * 3,
        out_shape=[jax.ShapeDtypeStruct((N, F), _f32)] * 3,
    )(s, dis, w, acc)


def _mid2_body(s_ref, dis_ref, txm2_ref, w_ref, acc_ref, tx_ref, u_ref,
               out_ref):
    dis = dis_ref[...]
    tx = -2.0 * dis * _cat(s_ref) - txm2_ref[...]
    tx_ref[...] = tx
    u_ref[...] = dis * tx
    out_ref[...] = acc_ref[...] + jnp.dot(tx, w_ref[...],
                                          preferred_element_type=_f32)


def _mid2(s, dis, txm2, w, acc):
    return pl.pallas_call(
        _mid2_body, grid=_GRID,
        in_specs=[_s_spec, _row_spec, _row_spec, _w_spec, _row_spec],
        out_specs=[_row_spec] * 3,
        out_shape=[jax.ShapeDtypeStruct((N, F), _f32)] * 3,
    )(s, dis, txm2, w, acc)


def _fin_body(s_ref, dis_ref, txm2_ref, w_ref, acc_ref, cb_ref, g_ref,
              be_ref, h_ref):
    tx = -2.0 * dis_ref[...] * _cat(s_ref) - txm2_ref[...]
    acc = acc_ref[...] + jnp.dot(tx, w_ref[...], preferred_element_type=_f32)
    h_ref[...] = jnp.maximum((acc + cb_ref[...]) * BNS * g_ref[...]
                             + be_ref[...], 0.0)


def _fin(s, dis, txm2, w, acc, cb, g, be):
    return pl.pallas_call(
        _fin_body, grid=_GRID,
        in_specs=[_s_spec, _row_spec, _row_spec, _w_spec, _row_spec,
                  _b_spec, _b_spec, _b_spec],
        out_specs=_row_spec,
        out_shape=jax.ShapeDtypeStruct((N, F), _f32),
    )(s, dis, txm2, w, acc, cb, g, be)


def _fin3_body(s_ref, dis_ref, txm2_ref, w_ref, acc_ref, cb_ref, g_ref,
               be_ref, hw_ref, hb_ref, o_ref):
    tx = -2.0 * dis_ref[...] * _cat(s_ref) - txm2_ref[...]
    acc = acc_ref[...] + jnp.dot(tx, w_ref[...], preferred_element_type=_f32)
    h = jnp.maximum((acc + cb_ref[...]) * BNS * g_ref[...] + be_ref[...], 0.0)
    o_ref[...] = jnp.dot(h, hw_ref[...], preferred_element_type=_f32) \
        + hb_ref[...]


def _fin3(s, dis, txm2, w, acc, cb, g, be, hw, hb):
    return pl.pallas_call(
        _fin3_body, grid=_GRID,
        in_specs=[_s_spec, _row_spec, _row_spec, _w_spec, _row_spec,
                  _b_spec, _b_spec, _b_spec, _w_spec, _b_spec],
        out_specs=_row_spec,
        out_shape=jax.ShapeDtypeStruct((N, F), _f32),
    )(s, dis, txm2, w, acc, cb, g, be, hw, hb)


def _layer(h, rc, dis, w, cb, g, be, head=None):
    u = _preu(h, dis)
    s = _lap(rc, u)
    acc = _mm(h, w[0])
    tx1, u = _midu1(s, dis)
    s = _lap(rc, u)
    acc = _mma(acc, tx1, w[1])
    tx2, u = _midu2(s, dis, h)
    s = _lap(rc, u)
    acc = _mma(acc, tx2, w[2])
    if head is None:
        return _fin(s, dis, tx1, w[3], acc, cb, g, be)
    return _fin3(s, dis, tx1, w[3], acc, cb, g, be, head[0], head[1])


def kernel(x, ei, W1, cb1, W2, cb2, W3, cb3, g1, be1, g2, be2, g3, be3,
           headW, headb):
    rows = ei[0].reshape(NW, NCH, C)
    cols = ei[1].reshape(NW, NCH, C)
    ei2 = jnp.stack([rows, cols], axis=2)
    deg_s = _deg(rows)
    dis = _degfin(deg_s)
    r2 = lambda v: v.reshape(1, F)
    h = _layer(x, ei2, dis, W1, r2(cb1), r2(g1), r2(be1))
    h = _layer(h, ei2, dis, W2, r2(cb2), r2(g2), r2(be2))
    return _layer(h, ei2, dis, W3, r2(cb3), r2(g3), r2(be3),
                  head=(headW, r2(headb)))


# speedup vs baseline: 1.0657x; 1.0009x over previous
import math

import jax
import jax.numpy as jnp
from jax import lax
from jax.experimental import pallas as pl
from jax.experimental.pallas import tpu as pltpu
from jax.experimental.pallas import tpu_sc as plsc

N = 10000
E = 320000
F = 128
NC = 2
NS = 16
NW = NC * NS
C = 125
NCH = (E // NW) // C
GR = 80
NGRP = N // GR
NGT = -(-NGRP // NS)
DEGW = 128
NBUF = 2
RB = 400
BNS = 1.0 / math.sqrt(1.0 + 1e-5)


def _fill2d(ref, nrows, ncols, value):
    v = jnp.full((16,), value, jnp.float32)

    def body(j, carry):
        for k in range(ncols // 16):
            ref[j, pl.ds(k * 16, 16)] = v
        return carry

    lax.fori_loop(0, nrows, body, 0)


def _deg_body(rows_hbm, out_hbm, rows_v, obuf, acc_sh, *ssems):
    cid = lax.axis_index("c")
    sid = lax.axis_index("s")
    wid = sid * NC + cid
    pltpu.sync_copy(rows_hbm.at[wid], rows_v)
    _fill2d(obuf, C, DEGW, 0.0)

    def zbody(t, carry):
        g = sid + NS * t

        @pl.when(g < NGRP)
        def _():
            pltpu.sync_copy(obuf.at[pl.ds(0, GR)], acc_sh.at[pl.ds(g * GR, GR)])

        return carry

    lax.fori_loop(0, NGT, zbody, 0)
    _fill2d(obuf, C, DEGW, 1.0)
    plsc.subcore_barrier()

    def body(t, carry):
        j0 = NBUF * t
        for k in range(NBUF):
            j = j0 + k

            @pl.when(t > 0)
            def _(k=k):
                pltpu.make_async_copy(obuf, acc_sh.at[rows_v.at[j - NBUF]],
                                      ssems[k]).wait()

            pltpu.async_copy(obuf, acc_sh.at[rows_v.at[j]], ssems[k],
                             add=True)
        return carry

    lax.fori_loop(0, NCH // NBUF, body, 0)
    for k in range(NBUF):
        pltpu.make_async_copy(obuf, acc_sh.at[rows_v.at[NCH - NBUF + k]],
                              ssems[k]).wait()
    plsc.subcore_barrier()

    def obody(t, carry):
        g = sid + NS * t

        @pl.when(g < NGRP)
        def _():
            pltpu.sync_copy(acc_sh.at[pl.ds(g * GR, GR)],
                            out_hbm.at[cid, pl.ds(g * GR, GR)])

        return carry

    lax.fori_loop(0, NGT, obody, 0)


_deg = pl.kernel(
    _deg_body,
    out_type=jax.ShapeDtypeStruct((NC, N, DEGW), jnp.float32),
    mesh=plsc.VectorSubcoreMesh(core_axis_name="c", subcore_axis_name="s"),
    scratch_types=[
        pltpu.VMEM((NCH, C), jnp.int32),
        pltpu.VMEM((C, DEGW), jnp.float32),
        pltpu.VMEM_SHARED((N, DEGW), jnp.float32),
    ] + [pltpu.SemaphoreType.DMA] * NBUF,
)


def _lap_body(ei2_hbm, u_hbm, out_hbm, ibuf, gb, acc_sh, *sems):
    isems = sems[:NBUF]
    gsems = sems[NBUF:]
    cid = lax.axis_index("c")
    sid = lax.axis_index("s")
    wid = sid * NC + cid
    eh = ei2_hbm.at[wid]
    _fill2d(gb.at[0], C, F, 0.0)

    def zbody(t, carry):
        g = sid + NS * t

        @pl.when(g < NGRP)
        def _():
            pltpu.sync_copy(gb.at[0, pl.ds(0, GR)], acc_sh.at[pl.ds(g * GR, GR)])

        return carry

    lax.fori_loop(0, NGT, zbody, 0)
    for b in range(NBUF):
        pltpu.async_copy(eh.at[b], ibuf.at[b], isems[b])
    plsc.subcore_barrier()

    def body(t, carry):
        j0 = NBUF * t
        for b in range(NBUF):
            j = j0 + b
            pltpu.make_async_copy(eh.at[j], ibuf.at[b], isems[b]).wait()
            pltpu.async_copy(u_hbm.at[ibuf.at[b, 1]], gb.at[b], gsems[b])
        for b in range(NBUF):
            j = j0 + b
            pltpu.make_async_copy(u_hbm.at[ibuf.at[b, 1]], gb.at[b],
                                  gsems[b]).wait()
            pltpu.sync_copy(gb.at[b], acc_sh.at[ibuf.at[b, 0]], add=True)

            @pl.when(j + NBUF < NCH)
            def _(j=j, b=b):
                pltpu.async_copy(eh.at[j + NBUF], ibuf.at[b], isems[b])

        return carry

    lax.fori_loop(0, NCH // NBUF, body, 0)
    plsc.subcore_barrier()

    def obody(t, carry):
        g = sid + NS * t

        @pl.when(g < NGRP)
        def _():
            pltpu.sync_copy(acc_sh.at[pl.ds(g * GR, GR)],
                            out_hbm.at[cid, pl.ds(g * GR, GR)])

        return carry

    lax.fori_loop(0, NGT, obody, 0)


_lap = pl.kernel(
    _lap_body,
    out_type=jax.ShapeDtypeStruct((NC, N, F), jnp.float32),
    mesh=plsc.VectorSubcoreMesh(core_axis_name="c", subcore_axis_name="s"),
    scratch_types=[
        pltpu.VMEM((NBUF, 2, C), jnp.int32),
        pltpu.VMEM((NBUF, C, F), jnp.float32),
        pltpu.VMEM_SHARED((N, F), jnp.float32),
    ] + [pltpu.SemaphoreType.DMA] * (2 * NBUF),
)


_row_spec = pl.BlockSpec((RB, F), lambda i: (i, 0))
_s_spec = pl.BlockSpec((NC, RB, F), lambda i: (0, i, 0))
_w_spec = pl.BlockSpec((F, F), lambda i: (0, 0))
_b_spec = pl.BlockSpec((1, F), lambda i: (0, 0))
_GRID = (N // RB,)
_f32 = jnp.float32


def _cat(s_ref):
    return s_ref[0] + s_ref[1]


def _degfin_body(s_ref, dis_ref):
    d = s_ref[0, :, 0:1] + s_ref[1, :, 0:1]
    dis = jnp.where(d > 0, lax.rsqrt(jnp.maximum(d, 1.0)), 0.0)
    dis_ref[...] = jnp.broadcast_to(dis, dis_ref.shape)


def _degfin(deg_s):
    return pl.pallas_call(
        _degfin_body,
        out_shape=jax.ShapeDtypeStruct((N, F), _f32),
    )(deg_s)


def _preu_body(h_ref, dis_ref, u_ref):
    u_ref[...] = dis_ref[...] * h_ref[...]


def _preu(h, dis):
    return pl.pallas_call(
        _preu_body, grid=_GRID,
        in_specs=[_row_spec, _row_spec],
        out_specs=_row_spec,
        out_shape=jax.ShapeDtypeStruct((N, F), _f32),
    )(h, dis)


def _mm_body(h_ref, w_ref, acc_ref):
    acc_ref[...] = jnp.dot(h_ref[...], w_ref[...], preferred_element_type=_f32)


def _mm(h, w):
    return pl.pallas_call(
        _mm_body, grid=_GRID,
        in_specs=[_row_spec, _w_spec],
        out_specs=_row_spec,
        out_shape=jax.ShapeDtypeStruct((N, F), _f32),
    )(h, w)


def _mma_body(acc_ref, tx_ref, w_ref, out_ref):
    out_ref[...] = acc_ref[...] + jnp.dot(tx_ref[...], w_ref[...],
                                          preferred_element_type=_f32)


def _mma(acc, tx, w):
    return pl.pallas_call(
        _mma_body, grid=_GRID,
        in_specs=[_row_spec, _row_spec, _w_spec],
        out_specs=_row_spec,
        out_shape=jax.ShapeDtypeStruct((N, F), _f32),
    )(acc, tx, w)


def _midu1_body(s_ref, dis_ref, tx_ref, u_ref):
    dis = dis_ref[...]
    tx = -dis * _cat(s_ref)
    tx_ref[...] = tx
    u_ref[...] = dis * tx


def _midu1(s, dis):
    return pl.pallas_call(
        _midu1_body, grid=_GRID,
        in_specs=[_s_spec, _row_spec],
        out_specs=[_row_spec] * 2,
        out_shape=[jax.ShapeDtypeStruct((N, F), _f32)] * 2,
    )(s, dis)


def _midu2_body(s_ref, dis_ref, txm2_ref, tx_ref, u_ref):
    dis = dis_ref[...]
    tx = -2.0 * dis * _cat(s_ref) - txm2_ref[...]
    tx_ref[...] = tx
    u_ref[...] = dis * tx


def _midu2(s, dis, txm2):
    return pl.pallas_call(
        _midu2_body, grid=_GRID,
        in_specs=[_s_spec, _row_spec, _row_spec],
        out_specs=[_row_spec] * 2,
        out_shape=[jax.ShapeDtypeStruct((N, F), _f32)] * 2,
    )(s, dis, txm2)


def _fin_body(s_ref, dis_ref, txm2_ref, w_ref, acc_ref, cb_ref, g_ref,
              be_ref, h_ref):
    tx = -2.0 * dis_ref[...] * _cat(s_ref) - txm2_ref[...]
    acc = acc_ref[...] + jnp.dot(tx, w_ref[...], preferred_element_type=_f32)
    h_ref[...] = jnp.maximum((acc + cb_ref[...]) * BNS * g_ref[...]
                             + be_ref[...], 0.0)


def _fin(s, dis, txm2, w, acc, cb, g, be):
    return pl.pallas_call(
        _fin_body, grid=_GRID,
        in_specs=[_s_spec, _row_spec, _row_spec, _w_spec, _row_spec,
                  _b_spec, _b_spec, _b_spec],
        out_specs=_row_spec,
        out_shape=jax.ShapeDtypeStruct((N, F), _f32),
    )(s, dis, txm2, w, acc, cb, g, be)


def _fin3_body(s_ref, dis_ref, txm2_ref, w_ref, acc_ref, cb_ref, g_ref,
               be_ref, hw_ref, hb_ref, o_ref):
    tx = -2.0 * dis_ref[...] * _cat(s_ref) - txm2_ref[...]
    acc = acc_ref[...] + jnp.dot(tx, w_ref[...], preferred_element_type=_f32)
    h = jnp.maximum((acc + cb_ref[...]) * BNS * g_ref[...] + be_ref[...], 0.0)
    o_ref[...] = jnp.dot(h, hw_ref[...], preferred_element_type=_f32) \
        + hb_ref[...]


def _fin3(s, dis, txm2, w, acc, cb, g, be, hw, hb):
    return pl.pallas_call(
        _fin3_body, grid=_GRID,
        in_specs=[_s_spec, _row_spec, _row_spec, _w_spec, _row_spec,
                  _b_spec, _b_spec, _b_spec, _w_spec, _b_spec],
        out_specs=_row_spec,
        out_shape=jax.ShapeDtypeStruct((N, F), _f32),
    )(s, dis, txm2, w, acc, cb, g, be, hw, hb)


def _layer(h, rc, dis, w, cb, g, be, head=None):
    u = _preu(h, dis)
    s = _lap(rc, u)
    acc = _mm(h, w[0])
    tx1, u = _midu1(s, dis)
    s = _lap(rc, u)
    acc = _mma(acc, tx1, w[1])
    tx2, u = _midu2(s, dis, h)
    s = _lap(rc, u)
    acc = _mma(acc, tx2, w[2])
    if head is None:
        return _fin(s, dis, tx1, w[3], acc, cb, g, be)
    return _fin3(s, dis, tx1, w[3], acc, cb, g, be, head[0], head[1])


def kernel(x, ei, W1, cb1, W2, cb2, W3, cb3, g1, be1, g2, be2, g3, be3,
           headW, headb):
    rows = ei[0].reshape(NW, NCH, C)
    cols = ei[1].reshape(NW, NCH, C)
    ei2 = jnp.stack([rows, cols], axis=2)
    deg_s = _deg(rows)
    dis = _degfin(deg_s)
    r2 = lambda v: v.reshape(1, F)
    h = _layer(x, ei2, dis, W1, r2(cb1), r2(g1), r2(be1))
    h = _layer(h, ei2, dis, W2, r2(cb2), r2(g2), r2(be2))
    return _layer(h, ei2, dis, W3, r2(cb3), r2(g3), r2(be3),
                  head=(headW, r2(headb)))
